# baseline (speedup 1.0000x reference)
import functools

import jax
import jax.numpy as jnp
from jax import lax
from jax.experimental import pallas as pl
from jax.experimental.pallas import tpu as pltpu
from jax.experimental.pallas import tpu_sc as plsc

B, D, R = 16384, 128, 1000
NC, NS = 2, 16
NW = NC * NS
ROWS_W = B // NW
RC = 128
NCH = ROWS_W // RC
NG = RC // 16
NBUF = 2


def _tree_sum(vals):
    while len(vals) > 1:
        vals = [a + b for a, b in zip(vals[::2], vals[1::2])]
    return vals[0]


def _sc_body(subj_hbm, obj_hbm, rel_hbm, table_hbm, out_hbm,
             idx_v, s_v, o_v, r_v, out_v, scr_v, tbl_sp, sem_s, sem_o, sem_r):
    wid = lax.axis_index("s") * NC + lax.axis_index("c")
    base = wid * ROWS_W
    lanes = lax.iota(jnp.int32, 16)

    @pl.when(lax.axis_index("s") == 0)
    def _():
        pltpu.sync_copy(table_hbm, tbl_sp)

    pltpu.sync_copy(rel_hbm.at[pl.ds(base, ROWS_W)], idx_v)
    plsc.subcore_barrier()

    def start_chunk(ci, buf):
        row0 = base + ci * RC
        pltpu.async_copy(tbl_sp.at[idx_v.at[pl.ds(ci * RC, RC)]],
                         r_v.at[buf], sem_r)
        pltpu.async_copy(subj_hbm.at[pl.ds(row0, RC)], s_v.at[buf], sem_s)
        pltpu.async_copy(obj_hbm.at[pl.ds(row0, RC)], o_v.at[buf], sem_o)

    def wait_chunk(ci, buf):
        pltpu.make_async_copy(tbl_sp.at[idx_v.at[pl.ds(ci * RC, RC)]],
                              r_v.at[buf], sem_r).wait()
        pltpu.make_async_copy(subj_hbm.at[pl.ds(0, RC)], s_v.at[buf], sem_s).wait()
        pltpu.make_async_copy(obj_hbm.at[pl.ds(0, RC)], o_v.at[buf], sem_o).wait()

    def compute_chunk(ci, buf):
        row0 = base + ci * RC
        sb, ob, rb = s_v.at[buf], o_v.at[buf], r_v.at[buf]

        def zero_body(g):
            out_v[pl.ds(g * 16, 16)] = jnp.zeros((16,), jnp.float32)

        plsc.parallel_loop(0, NG, 1)(zero_body)

        def row_body(r):
            acc = (sb[r, pl.ds(0, 16)] * rb[r, pl.ds(0, 16)]
                   * ob[r, pl.ds(0, 16)])
            for j in range(1, D // 16):
                acc += (sb[r, pl.ds(16 * j, 16)]
                        * rb[r, pl.ds(16 * j, 16)]
                        * ob[r, pl.ds(16 * j, 16)])
            plsc.addupdate_scatter(out_v, [lanes * 0 + r], acc)

        plsc.parallel_loop(0, RC, 1, unroll=4)(row_body)
        pltpu.sync_copy(out_v, out_hbm.at[pl.ds(row0, RC)])

    start_chunk(0, 0)

    def chunk_body(ci, _):
        buf = lax.rem(ci, NBUF)

        @pl.when(ci + 1 < NCH)
        def _():
            start_chunk(ci + 1, lax.rem(ci + 1, NBUF))

        wait_chunk(ci, buf)
        compute_chunk(ci, buf)
        return 0

    lax.fori_loop(0, NCH, chunk_body, 0)


@jax.jit
def _scores_sc(subject_embeddings, object_embeddings, relations, relation_table):
    mesh = plsc.VectorSubcoreMesh(core_axis_name="c", subcore_axis_name="s")
    f = functools.partial(
        pl.kernel,
        out_type=jax.ShapeDtypeStruct((B,), jnp.float32),
        mesh=mesh,
        scratch_types=[
            pltpu.VMEM((ROWS_W,), jnp.int32),
            pltpu.VMEM((NBUF, RC, D), jnp.float32),
            pltpu.VMEM((NBUF, RC, D), jnp.float32),
            pltpu.VMEM((NBUF, RC, D), jnp.float32),
            pltpu.VMEM((RC,), jnp.float32),
            pltpu.VMEM((NG * 256,), jnp.float32),
            pltpu.VMEM_SHARED((R, D), jnp.float32),
            pltpu.SemaphoreType.DMA,
            pltpu.SemaphoreType.DMA,
            pltpu.SemaphoreType.DMA,
        ],
        compiler_params=pltpu.CompilerParams(needs_layout_passes=False),
    )(_sc_body)
    return f(subject_embeddings, object_embeddings, relations, relation_table)


def kernel(subject_embeddings, object_embeddings, relations, relation_table):
    scores = _scores_sc(subject_embeddings, object_embeddings,
                        relations.astype(jnp.int32), relation_table)
    return scores.reshape(B, 1)


# speedup vs baseline: 1.2400x; 1.2400x over previous
import functools

import jax
import jax.numpy as jnp
from jax import lax
from jax.experimental import pallas as pl
from jax.experimental.pallas import tpu as pltpu
from jax.experimental.pallas import tpu_sc as plsc

B, D, R = 16384, 128, 1000
NC, NS = 2, 16
NW = NC * NS
ROWS_W = B // NW
RC = 128
NCH = ROWS_W // RC
NG = RC // 16
NBUF = 2


def _tree_sum(vals):
    while len(vals) > 1:
        vals = [a + b for a, b in zip(vals[::2], vals[1::2])]
    return vals[0]


def _sc_body(subj_hbm, obj_hbm, rel_hbm, table_hbm, out_hbm,
             idx_v, s_v, o_v, r_v, out_v, scr_v, tbl_sp, sem_s, sem_o, sem_r):
    wid = lax.axis_index("s") * NC + lax.axis_index("c")
    base = wid * ROWS_W
    lanes = lax.iota(jnp.int32, 16)

    @pl.when(lax.axis_index("s") == 0)
    def _():
        pltpu.sync_copy(table_hbm, tbl_sp)

    pltpu.sync_copy(rel_hbm.at[pl.ds(base, ROWS_W)], idx_v)
    plsc.subcore_barrier()

    def start_chunk(ci, buf):
        row0 = base + ci * RC
        pltpu.async_copy(tbl_sp.at[idx_v.at[pl.ds(ci * RC, RC)]],
                         r_v.at[buf], sem_r)
        pltpu.async_copy(subj_hbm.at[pl.ds(row0, RC)], s_v.at[buf], sem_s)
        pltpu.async_copy(obj_hbm.at[pl.ds(row0, RC)], o_v.at[buf], sem_o)

    def wait_chunk(ci, buf):
        pltpu.make_async_copy(tbl_sp.at[idx_v.at[pl.ds(ci * RC, RC)]],
                              r_v.at[buf], sem_r).wait()
        pltpu.make_async_copy(subj_hbm.at[pl.ds(0, RC)], s_v.at[buf], sem_s).wait()
        pltpu.make_async_copy(obj_hbm.at[pl.ds(0, RC)], o_v.at[buf], sem_o).wait()

    def compute_chunk(ci, buf):
        row0 = base + ci * RC
        sb, ob, rb = s_v.at[buf], o_v.at[buf], r_v.at[buf]

        def row_body(r):
            acc = (sb[r, pl.ds(0, 16)] * rb[r, pl.ds(0, 16)]
                   * ob[r, pl.ds(0, 16)])
            for j in range(1, D // 16):
                acc += (sb[r, pl.ds(16 * j, 16)]
                        * rb[r, pl.ds(16 * j, 16)]
                        * ob[r, pl.ds(16 * j, 16)])
            g = lax.div(r, 16)
            rr = lax.rem(r, 16)
            plsc.store_scatter(scr_v, [g * 256 + lanes * 16 + rr], acc)

        plsc.parallel_loop(0, RC, 1, unroll=2)(row_body)

        def reduce_body(g):
            sbase = g * 256
            res = _tree_sum([scr_v[pl.ds(sbase + c * 16, 16)]
                             for c in range(16)])
            out_v[pl.ds(g * 16, 16)] = res

        plsc.parallel_loop(0, NG, 1)(reduce_body)
        pltpu.sync_copy(out_v, out_hbm.at[pl.ds(row0, RC)])

    start_chunk(0, 0)

    def chunk_body(ci, _):
        buf = lax.rem(ci, NBUF)

        @pl.when(ci + 1 < NCH)
        def _():
            start_chunk(ci + 1, lax.rem(ci + 1, NBUF))

        wait_chunk(ci, buf)
        compute_chunk(ci, buf)
        return 0

    lax.fori_loop(0, NCH, chunk_body, 0)


@jax.jit
def _scores_sc(subject_embeddings, object_embeddings, relations, relation_table):
    mesh = plsc.VectorSubcoreMesh(core_axis_name="c", subcore_axis_name="s")
    f = functools.partial(
        pl.kernel,
        out_type=jax.ShapeDtypeStruct((B,), jnp.float32),
        mesh=mesh,
        scratch_types=[
            pltpu.VMEM((ROWS_W,), jnp.int32),
            pltpu.VMEM((NBUF, RC, D), jnp.float32),
            pltpu.VMEM((NBUF, RC, D), jnp.float32),
            pltpu.VMEM((NBUF, RC, D), jnp.float32),
            pltpu.VMEM((RC,), jnp.float32),
            pltpu.VMEM((NG * 256,), jnp.float32),
            pltpu.VMEM_SHARED((R, D), jnp.float32),
            pltpu.SemaphoreType.DMA,
            pltpu.SemaphoreType.DMA,
            pltpu.SemaphoreType.DMA,
        ],
        compiler_params=pltpu.CompilerParams(needs_layout_passes=False),
    )(_sc_body)
    return f(subject_embeddings, object_embeddings, relations, relation_table)


def kernel(subject_embeddings, object_embeddings, relations, relation_table):
    scores = _scores_sc(subject_embeddings, object_embeddings,
                        relations.astype(jnp.int32), relation_table)
    return scores.reshape(B, 1)
